# baseline (speedup 1.0000x reference)
import functools

import jax
import jax.numpy as jnp
from jax import lax
from jax.experimental import pallas as pl
from jax.experimental.pallas import tpu as pltpu
from jax.experimental.pallas import tpu_sc as plsc

_SMOOTH = 1e-05
_NC, _NS, _L = 2, 16, 16
_NW = _NC * _NS

_LN2 = 0.6931471805599453


def _ln(v):
    bits = plsc.bitcast(v, jnp.int32)
    e = lax.shift_right_logical(bits, 23) - 127
    mbits = lax.bitwise_or(lax.bitwise_and(bits, 0x007FFFFF), 0x3F800000)
    m = plsc.bitcast(mbits, jnp.float32)
    r = (m - 1.0) / (m + 1.0)
    r2 = r * r
    p = jnp.float32(1.0 / 9.0)
    p = p * r2 + jnp.float32(1.0 / 7.0)
    p = p * r2 + jnp.float32(1.0 / 5.0)
    p = p * r2 + jnp.float32(1.0 / 3.0)
    p = p * r2 + jnp.float32(1.0)
    ln_m = 2.0 * r * p
    return e.astype(jnp.float32) * jnp.float32(_LN2) + ln_m


def _sc_body(chunk, nchunk, hw, logit_hbm, tgt_hbm, alpha_hbm, loss_hbm, cnt_hbm,
             ch_v, t_v, alpha_v, out_v):
    wid = lax.axis_index("s") * _NC + lax.axis_index("c")
    per_w = chunk * nchunk
    sub_per_batch = hw // per_w
    b = wid // sub_per_batch
    off0 = (wid % sub_per_batch) * per_w

    pltpu.sync_copy(alpha_hbm, alpha_v)

    def chunk_body(k, carry):
        acc, cnt = carry
        start = off0 + k * chunk
        for c in range(5):
            pltpu.sync_copy(
                logit_hbm.at[pl.ds((b * 5 + c) * hw + start, chunk)],
                ch_v.at[pl.ds(c * chunk, chunk)],
            )
        pltpu.sync_copy(tgt_hbm.at[pl.ds(b * hw + start, chunk)], t_v)

        lane0 = lax.iota(jnp.int32, _L)

        def vec_body(i, vcarry):
            acc, cnt, col = vcarry
            t = t_v[pl.ds(i * _L, _L)]
            v = plsc.load_gather(ch_v, [t * chunk + col])
            v = jnp.minimum(jnp.maximum(v, jnp.float32(_SMOOTH)), jnp.float32(1.0))
            ln_v = _ln(v)
            a = plsc.load_gather(alpha_v, [t])
            acc = acc + a * (ln_v + jnp.float32(_SMOOTH))
            cnt = cnt + jnp.where(t > 0, jnp.float32(1.0), jnp.float32(0.0))
            return acc, cnt, col + _L

        acc, cnt, _ = lax.fori_loop(0, chunk // _L, vec_body, (acc, cnt, lane0))
        return acc, cnt

    zero = jnp.zeros((_L,), jnp.float32)
    acc, cnt = lax.fori_loop(0, nchunk, chunk_body, (zero, zero))

    out_v[...] = acc
    pltpu.sync_copy(out_v, loss_hbm.at[pl.ds(wid * _L, _L)])
    out_v[...] = cnt
    pltpu.sync_copy(out_v, cnt_hbm.at[pl.ds(wid * _L, _L)])


def kernel(logit, target, class_for_batch):
    B, C, H, W = logit.shape
    HW = H * W
    n = B * HW
    per_w = n // _NW
    chunk = 8192
    nchunk = per_w // chunk

    present = (jnp.arange(C)[:, None] == class_for_batch[None, :]).any(axis=1)
    alpha = jnp.where(present, 1.0, 0.0).astype(jnp.float32)
    alpha = alpha.at[0].set(0.0)
    alpha16 = jnp.zeros((_L,), jnp.float32).at[:C].set(alpha)

    lg = logit.reshape(-1)
    tg = target.reshape(-1)

    mesh = plsc.VectorSubcoreMesh(
        core_axis_name="c", subcore_axis_name="s", num_cores=_NC, num_subcores=_NS
    )
    loss_part, cnt_part = pl.kernel(
        functools.partial(_sc_body, chunk, nchunk, HW),
        out_type=[
            jax.ShapeDtypeStruct((_NW * _L,), jnp.float32),
            jax.ShapeDtypeStruct((_NW * _L,), jnp.float32),
        ],
        mesh=mesh,
        compiler_params=pltpu.CompilerParams(
            use_tc_tiling_on_sc=False, needs_layout_passes=False
        ),
        scratch_types=[
            pltpu.VMEM((C * chunk,), jnp.float32),
            pltpu.VMEM((chunk,), jnp.int32),
            pltpu.VMEM((_L,), jnp.float32),
            pltpu.VMEM((_L,), jnp.float32),
        ],
    )(lg, tg, alpha16)

    s = -jnp.sum(loss_part)
    pos = jnp.sum(cnt_part)
    return jnp.where(pos > 0, s / pos, s / jnp.float32(n))


# speedup vs baseline: 1.3958x; 1.3958x over previous
import functools

import jax
import jax.numpy as jnp
from jax import lax
from jax.experimental import pallas as pl
from jax.experimental.pallas import tpu as pltpu
from jax.experimental.pallas import tpu_sc as plsc

_SMOOTH = 1e-05
_NC, _NS, _L = 2, 16, 16
_NW = _NC * _NS

_LN2 = 0.6931471805599453
_C = (-2.0990698, 4.204513, -3.6488032, 2.2311249,
      -0.85552603, 0.18497244, -0.0172078)


def _ln(v):
    bits = plsc.bitcast(v, jnp.int32)
    e = lax.shift_right_logical(bits, 23) - 127
    mbits = lax.bitwise_or(lax.bitwise_and(bits, 0x007FFFFF), 0x3F800000)
    m = plsc.bitcast(mbits, jnp.float32)
    p = jnp.float32(_C[6])
    for k in (5, 4, 3, 2, 1, 0):
        p = p * m + jnp.float32(_C[k])
    return e.astype(jnp.float32) * jnp.float32(_LN2) + p


def _sc_body(rows_w, rch, logit_hbm, tgt_hbm, alpha_hbm, loss_hbm, cnt_hbm,
             ch_v, t_v, alpha_v, out_v):
    wid = lax.axis_index("s") * _NC + lax.axis_index("c")
    H, W = tgt_hbm.shape[1], tgt_hbm.shape[2]
    sub_per_batch = H // rows_w
    b = wid // sub_per_batch
    row0 = (wid % sub_per_batch) * rows_w

    pltpu.sync_copy(alpha_hbm, alpha_v)

    def chunk_body(k, carry):
        acc, cnt = carry
        r = row0 + k * rch
        for c in range(5):
            pltpu.sync_copy(logit_hbm.at[b, c, pl.ds(r, rch), :], ch_v.at[c])
        pltpu.sync_copy(tgt_hbm.at[b, pl.ds(r, rch), :], t_v)

        lane = lax.iota(jnp.int32, _L)

        def row_body(i, rcarry):
            acc, cnt = rcarry
            rowsplat = jnp.full((_L,), i, jnp.int32)

            def col_body(j, ccarry):
                acc, cnt, col = ccarry
                t = plsc.load_gather(t_v, [rowsplat, col])
                v = plsc.load_gather(ch_v, [t, rowsplat, col])
                v = jnp.maximum(v, jnp.float32(_SMOOTH))
                ln_v = _ln(v)
                a = plsc.load_gather(alpha_v, [t])
                acc = acc + a * (ln_v + jnp.float32(_SMOOTH))
                cnt = cnt + jnp.where(t > 0, jnp.float32(1.0), jnp.float32(0.0))
                return acc, cnt, col + _L

            acc, cnt, _ = lax.fori_loop(0, W // _L, col_body, (acc, cnt, lane))
            return acc, cnt

        return lax.fori_loop(0, rch, row_body, (acc, cnt))

    zero = jnp.zeros((_L,), jnp.float32)
    acc, cnt = lax.fori_loop(0, (rows_w // rch), chunk_body, (zero, zero))

    out_v[...] = acc
    pltpu.sync_copy(out_v, loss_hbm.at[pl.ds(wid * _L, _L)])
    out_v[...] = cnt
    pltpu.sync_copy(out_v, cnt_hbm.at[pl.ds(wid * _L, _L)])


def kernel(logit, target, class_for_batch):
    B, C, H, W = logit.shape
    n = B * H * W
    rows_w = (B * H) // _NW
    rch = 16

    present = (jnp.arange(C)[:, None] == class_for_batch[None, :]).any(axis=1)
    alpha = jnp.where(present, 1.0, 0.0).astype(jnp.float32)
    alpha = alpha.at[0].set(0.0)
    alpha16 = jnp.zeros((_L,), jnp.float32).at[:C].set(alpha)

    tg = target.reshape(B, H, W)

    mesh = plsc.VectorSubcoreMesh(
        core_axis_name="c", subcore_axis_name="s", num_cores=_NC, num_subcores=_NS
    )
    loss_part, cnt_part = pl.kernel(
        functools.partial(_sc_body, rows_w, rch),
        out_type=[
            jax.ShapeDtypeStruct((_NW * _L,), jnp.float32),
            jax.ShapeDtypeStruct((_NW * _L,), jnp.float32),
        ],
        mesh=mesh,
        compiler_params=pltpu.CompilerParams(
            use_tc_tiling_on_sc=True, needs_layout_passes=False
        ),
        scratch_types=[
            pltpu.VMEM((C, rch, W), jnp.float32),
            pltpu.VMEM((rch, W), jnp.int32),
            pltpu.VMEM((_L,), jnp.float32),
            pltpu.VMEM((_L,), jnp.float32),
        ],
    )(logit, tg, alpha16)

    s = -jnp.sum(loss_part)
    pos = jnp.sum(cnt_part)
    return jnp.where(pos > 0, s / pos, s / jnp.float32(n))


# speedup vs baseline: 2.3142x; 1.6580x over previous
import functools

import jax
import jax.numpy as jnp
from jax import lax
from jax.experimental import pallas as pl
from jax.experimental.pallas import tpu as pltpu
from jax.experimental.pallas import tpu_sc as plsc

_SMOOTH = 1e-05
_NC, _NS, _L = 2, 16, 16
_NW = _NC * _NS

_LN2 = 0.6931471805599453
_C = (-2.0990698, 4.204513, -3.6488032, 2.2311249,
      -0.85552603, 0.18497244, -0.0172078)


def _ln(v):
    bits = plsc.bitcast(v, jnp.int32)
    e = lax.shift_right_logical(bits, 23) - 127
    mbits = lax.bitwise_or(lax.bitwise_and(bits, 0x007FFFFF), 0x3F800000)
    m = plsc.bitcast(mbits, jnp.float32)
    p = jnp.float32(_C[6])
    for k in (5, 4, 3, 2, 1, 0):
        p = p * m + jnp.float32(_C[k])
    return e.astype(jnp.float32) * jnp.float32(_LN2) + p


def _sc_body(rows_w, rch, logit_hbm, tgt_hbm, alpha_hbm, loss_hbm, cnt_hbm,
             ch_v, t_v, alpha_v, out_v, sem_a, sem_b):
    wid = lax.axis_index("s") * _NC + lax.axis_index("c")
    H, W = tgt_hbm.shape[1], tgt_hbm.shape[2]
    nch = rows_w // rch
    sub_per_batch = H // rows_w
    b = wid // sub_per_batch
    row0 = (wid % sub_per_batch) * rows_w
    sems = (sem_a, sem_b)

    pltpu.sync_copy(alpha_hbm, alpha_v)

    def start(k, slot):
        r = row0 + k * rch
        hs = []
        for c in range(5):
            hs.append(pltpu.async_copy(
                logit_hbm.at[b, c, pl.ds(r, rch), :], ch_v.at[slot, c],
                sems[slot]))
        hs.append(pltpu.async_copy(
            tgt_hbm.at[b, pl.ds(r, rch), :], t_v.at[slot], sems[slot]))
        return hs

    def compute(slot, acc, cnt):
        lane = lax.iota(jnp.int32, _L)

        def row_body(i, rcarry):
            acc, cnt = rcarry
            rowsplat = jnp.full((_L,), i, jnp.int32)

            def col_body(j, ccarry):
                acc, cnt, col = ccarry
                t = plsc.load_gather(t_v.at[slot], [rowsplat, col])
                v = plsc.load_gather(ch_v.at[slot], [t, rowsplat, col])
                v = jnp.maximum(v, jnp.float32(_SMOOTH))
                ln_v = _ln(v)
                a = plsc.load_gather(alpha_v, [t])
                acc = acc + a * (ln_v + jnp.float32(_SMOOTH))
                cnt = cnt + jnp.where(t > 0, jnp.float32(1.0), jnp.float32(0.0))
                return acc, cnt, col + _L

            acc, cnt, _ = lax.fori_loop(0, W // _L, col_body, (acc, cnt, lane))
            return acc, cnt

        return lax.fori_loop(0, rch, row_body, (acc, cnt))

    acc = jnp.zeros((_L,), jnp.float32)
    cnt = jnp.zeros((_L,), jnp.float32)
    pending = start(0, 0)
    for k in range(nch):
        slot = k % 2
        for h in pending:
            h.wait()
        if k + 1 < nch:
            pending = start(k + 1, 1 - slot)
        acc, cnt = compute(slot, acc, cnt)

    out_v[...] = acc
    pltpu.sync_copy(out_v, loss_hbm.at[pl.ds(wid * _L, _L)])
    out_v[...] = cnt
    pltpu.sync_copy(out_v, cnt_hbm.at[pl.ds(wid * _L, _L)])


def kernel(logit, target, class_for_batch):
    B, C, H, W = logit.shape
    n = B * H * W
    rows_w = (B * H) // _NW
    rch = 16

    present = (jnp.arange(C)[:, None] == class_for_batch[None, :]).any(axis=1)
    alpha = jnp.where(present, 1.0, 0.0).astype(jnp.float32)
    alpha = alpha.at[0].set(0.0)
    alpha16 = jnp.zeros((_L,), jnp.float32).at[:C].set(alpha)

    tg = target.reshape(B, H, W)

    mesh = plsc.VectorSubcoreMesh(
        core_axis_name="c", subcore_axis_name="s", num_cores=_NC, num_subcores=_NS
    )
    loss_part, cnt_part = pl.kernel(
        functools.partial(_sc_body, rows_w, rch),
        out_type=[
            jax.ShapeDtypeStruct((_NW * _L,), jnp.float32),
            jax.ShapeDtypeStruct((_NW * _L,), jnp.float32),
        ],
        mesh=mesh,
        compiler_params=pltpu.CompilerParams(
            use_tc_tiling_on_sc=True, needs_layout_passes=False
        ),
        scratch_types=[
            pltpu.VMEM((2, C, rch, W), jnp.float32),
            pltpu.VMEM((2, rch, W), jnp.int32),
            pltpu.VMEM((_L,), jnp.float32),
            pltpu.VMEM((_L,), jnp.float32),
            pltpu.SemaphoreType.DMA,
            pltpu.SemaphoreType.DMA,
        ],
    )(logit, tg, alpha16)

    s = -jnp.sum(loss_part)
    pos = jnp.sum(cnt_part)
    return jnp.where(pos > 0, s / pos, s / jnp.float32(n))


# speedup vs baseline: 2.3586x; 1.0192x over previous
import functools

import jax
import jax.numpy as jnp
from jax import lax
from jax.experimental import pallas as pl
from jax.experimental.pallas import tpu as pltpu
from jax.experimental.pallas import tpu_sc as plsc

_SMOOTH = 1e-05
_NC, _NS, _L = 2, 16, 16
_NW = _NC * _NS

_LN2 = 0.6931471805599453
_C = (-1.9316664, 3.4982119, -2.420793, 1.1047965, -0.28062916, 0.030102247)


def _ln(v):
    bits = plsc.bitcast(v, jnp.int32)
    e = lax.shift_right_logical(bits, 23) - 127
    mbits = lax.bitwise_or(lax.bitwise_and(bits, 0x007FFFFF), 0x3F800000)
    m = plsc.bitcast(mbits, jnp.float32)
    p = jnp.float32(_C[5])
    for k in (4, 3, 2, 1, 0):
        p = p * m + jnp.float32(_C[k])
    return e.astype(jnp.float32) * jnp.float32(_LN2) + p


def _sc_body(rows_w, rch, logit_hbm, tgt_hbm, alpha_hbm, loss_hbm, cnt_hbm,
             ch_v, t_v, alpha_v, out_v, sem_a, sem_b):
    wid = lax.axis_index("s") * _NC + lax.axis_index("c")
    H, W = tgt_hbm.shape[1], tgt_hbm.shape[2]
    nch = rows_w // rch
    sub_per_batch = H // rows_w
    b = wid // sub_per_batch
    row0 = (wid % sub_per_batch) * rows_w
    sems = (sem_a, sem_b)

    pltpu.sync_copy(alpha_hbm, alpha_v)

    def start(k, slot):
        r = row0 + k * rch
        hs = []
        for c in range(5):
            hs.append(pltpu.async_copy(
                logit_hbm.at[b, c, pl.ds(r, rch), :], ch_v.at[slot, c],
                sems[slot]))
        hs.append(pltpu.async_copy(
            tgt_hbm.at[b, pl.ds(r, rch), :], t_v.at[slot], sems[slot]))
        return hs

    abits = alpha_v[...]
    unroll = 4

    def compute(slot, acc, cnt):
        lane = lax.iota(jnp.int32, _L)
        tsl = t_v.at[slot]
        csl = ch_v.at[slot]

        def row_body(i, rcarry):
            acc, cnt = rcarry
            rowsplat = jnp.full((_L,), i, jnp.int32)

            def col_body(j, ccarry):
                acc, cnt = ccarry
                for u in range(unroll):
                    col0 = (j * unroll + u) * _L
                    t = tsl[i, pl.ds(col0, _L)]
                    v = plsc.load_gather(csl, [t, rowsplat, lane + col0])
                    v = jnp.maximum(v, jnp.float32(_SMOOTH))
                    ln_v = _ln(v)
                    a = lax.bitwise_and(
                        lax.shift_right_logical(abits, t), 1
                    ).astype(jnp.float32)
                    acc = acc + a * (ln_v + jnp.float32(_SMOOTH))
                    cnt = cnt + jnp.where(t > 0, jnp.float32(1.0),
                                          jnp.float32(0.0))
                return acc, cnt

            return lax.fori_loop(0, W // (_L * unroll), col_body, (acc, cnt))

        return lax.fori_loop(0, rch, row_body, (acc, cnt))

    acc = jnp.zeros((_L,), jnp.float32)
    cnt = jnp.zeros((_L,), jnp.float32)
    pending = start(0, 0)
    for k in range(nch):
        slot = k % 2
        for h in pending:
            h.wait()
        if k + 1 < nch:
            pending = start(k + 1, 1 - slot)
        acc, cnt = compute(slot, acc, cnt)

    out_v[...] = acc
    pltpu.sync_copy(out_v, loss_hbm.at[pl.ds(wid * _L, _L)])
    out_v[...] = cnt
    pltpu.sync_copy(out_v, cnt_hbm.at[pl.ds(wid * _L, _L)])


def kernel(logit, target, class_for_batch):
    B, C, H, W = logit.shape
    n = B * H * W
    rows_w = (B * H) // _NW
    rch = 16

    present = (jnp.arange(C)[:, None] == class_for_batch[None, :]).any(axis=1)
    alpha = jnp.where(present, 1.0, 0.0).astype(jnp.float32)
    alpha = alpha.at[0].set(0.0)
    abits = jnp.sum(
        jnp.where(alpha > 0, (1 << jnp.arange(C)).astype(jnp.int32), 0)
    ).astype(jnp.int32)
    abits16 = jnp.full((_L,), abits, jnp.int32)

    tg = target.reshape(B, H, W)

    mesh = plsc.VectorSubcoreMesh(
        core_axis_name="c", subcore_axis_name="s", num_cores=_NC, num_subcores=_NS
    )
    loss_part, cnt_part = pl.kernel(
        functools.partial(_sc_body, rows_w, rch),
        out_type=[
            jax.ShapeDtypeStruct((_NW * _L,), jnp.float32),
            jax.ShapeDtypeStruct((_NW * _L,), jnp.float32),
        ],
        mesh=mesh,
        compiler_params=pltpu.CompilerParams(
            use_tc_tiling_on_sc=True, needs_layout_passes=False
        ),
        scratch_types=[
            pltpu.VMEM((2, C, rch, W), jnp.float32),
            pltpu.VMEM((2, rch, W), jnp.int32),
            pltpu.VMEM((_L,), jnp.int32),
            pltpu.VMEM((_L,), jnp.float32),
            pltpu.SemaphoreType.DMA,
            pltpu.SemaphoreType.DMA,
        ],
    )(logit, tg, abits16)

    s = -jnp.sum(loss_part)
    pos = jnp.sum(cnt_part)
    return jnp.where(pos > 0, s / pos, s / jnp.float32(n))


# speedup vs baseline: 2.6162x; 1.1092x over previous
import functools

import jax
import jax.numpy as jnp
from jax import lax
from jax.experimental import pallas as pl
from jax.experimental.pallas import tpu as pltpu
from jax.experimental.pallas import tpu_sc as plsc

_SMOOTH = 1e-05
_NC, _NS, _L = 2, 16, 16
_NW = _NC * _NS

_K1 = 0.6931471805599453 / (1 << 23)
_G = (-88.02955, 3.6034518e-08, -6.5948397e-15, 3.6661022e-22, -1.1079349e-29)


def _ln(v):
    bits = plsc.bitcast(v, jnp.int32)
    y1 = bits.astype(jnp.float32) * jnp.float32(_K1)
    mant = lax.bitwise_and(bits, 0x007FFFFF).astype(jnp.float32)
    p = jnp.float32(_G[4])
    for k in (3, 2, 1, 0):
        p = p * mant + jnp.float32(_G[k])
    return y1 + p


def _sc_body(rows_w, rch, logit_hbm, tgt_hbm, abits_hbm, loss_hbm, cnt_hbm,
             ch_v, t_v, abits_vm, out_v, sem_a, sem_b):
    wid = lax.axis_index("s") * _NC + lax.axis_index("c")
    H, W = tgt_hbm.shape[1], tgt_hbm.shape[2]
    C = logit_hbm.shape[1]
    nch = rows_w // rch
    sub_per_batch = H // rows_w
    b = wid // sub_per_batch
    row0 = (wid % sub_per_batch) * rows_w
    sems = (sem_a, sem_b)

    pltpu.sync_copy(abits_hbm, abits_vm)
    abits_v = abits_vm[...]

    def copies(chunk, slot):
        r = row0 + chunk * rch
        srcs = [logit_hbm.at[b, c, pl.ds(r, rch), :] for c in range(C)]
        srcs.append(tgt_hbm.at[b, pl.ds(r, rch), :])
        dsts = [ch_v.at[slot, c] for c in range(C)]
        dsts.append(t_v.at[slot])
        return [(s, d, sems[slot]) for s, d in zip(srcs, dsts)]

    def issue(chunk, slot):
        for s, d, sem in copies(chunk, slot):
            pltpu.async_copy(s, d, sem)

    def drain(chunk, slot):
        for s, d, sem in copies(chunk, slot):
            pltpu.make_async_copy(s, d, sem).wait()

    unroll = 4

    def compute(slot, acc, cnt):
        lane = lax.iota(jnp.int32, _L)
        tsl = t_v.at[slot]
        csl = ch_v.at[slot]

        def row_body(i, rcarry):
            acc, cnt = rcarry
            rowsplat = jnp.full((_L,), i, jnp.int32)

            def col_body(j, ccarry):
                acc, cnt = ccarry
                for u in range(unroll):
                    col0 = (j * unroll + u) * _L
                    t = tsl[i, pl.ds(col0, _L)]
                    v = plsc.load_gather(csl, [t, rowsplat, lane + col0])
                    v = jnp.maximum(v, jnp.float32(_SMOOTH))
                    ln_v = _ln(v)
                    a = lax.bitwise_and(
                        lax.shift_right_logical(abits_v, t), 1
                    ).astype(jnp.float32)
                    acc = acc + a * (ln_v + jnp.float32(_SMOOTH))
                    cnt = cnt + jnp.minimum(t, 1)
                return acc, cnt

            return lax.fori_loop(0, W // (_L * unroll), col_body, (acc, cnt))

        return lax.fori_loop(0, rch, row_body, (acc, cnt))

    acc = jnp.zeros((_L,), jnp.float32)
    cnt = jnp.zeros((_L,), jnp.int32)
    issue(0, 0)

    def pair_body(k2, carry):
        acc, cnt = carry
        c0 = 2 * k2
        drain(c0, 0)
        issue(c0 + 1, 1)
        acc, cnt = compute(0, acc, cnt)

        @pl.when(c0 + 2 < nch)
        def _():
            issue(c0 + 2, 0)

        drain(c0 + 1, 1)
        acc, cnt = compute(1, acc, cnt)
        return acc, cnt

    acc, cnt = lax.fori_loop(0, nch // 2, pair_body, (acc, cnt))

    out_v[...] = acc
    pltpu.sync_copy(out_v, loss_hbm.at[pl.ds(wid * _L, _L)])
    out_v[...] = cnt.astype(jnp.float32)
    pltpu.sync_copy(out_v, cnt_hbm.at[pl.ds(wid * _L, _L)])


def kernel(logit, target, class_for_batch):
    B, C, H, W = logit.shape
    n = B * H * W
    rows_w = (B * H) // _NW
    rch = 16

    present = (jnp.arange(C)[:, None] == class_for_batch[None, :]).any(axis=1)
    alpha = jnp.where(present, 1.0, 0.0).astype(jnp.float32)
    alpha = alpha.at[0].set(0.0)
    abits = jnp.sum(
        jnp.where(alpha > 0, (1 << jnp.arange(C)).astype(jnp.int32), 0)
    ).astype(jnp.int32)
    abits16 = jnp.full((_L,), abits, jnp.int32)

    tg = target.reshape(B, H, W)

    mesh = plsc.VectorSubcoreMesh(
        core_axis_name="c", subcore_axis_name="s", num_cores=_NC, num_subcores=_NS
    )
    loss_part, cnt_part = pl.kernel(
        functools.partial(_sc_body, rows_w, rch),
        out_type=[
            jax.ShapeDtypeStruct((_NW * _L,), jnp.float32),
            jax.ShapeDtypeStruct((_NW * _L,), jnp.float32),
        ],
        mesh=mesh,
        compiler_params=pltpu.CompilerParams(
            use_tc_tiling_on_sc=True, needs_layout_passes=False
        ),
        scratch_types=[
            pltpu.VMEM((2, C, rch, W), jnp.float32),
            pltpu.VMEM((2, rch, W), jnp.int32),
            pltpu.VMEM((_L,), jnp.int32),
            pltpu.VMEM((_L,), jnp.float32),
            pltpu.SemaphoreType.DMA,
            pltpu.SemaphoreType.DMA,
        ],
    )(logit, tg, abits16)

    s = -jnp.sum(loss_part)
    pos = jnp.sum(cnt_part)
    return jnp.where(pos > 0, s / pos, s / jnp.float32(n))


# speedup vs baseline: 2.6281x; 1.0045x over previous
import functools

import jax
import jax.numpy as jnp
from jax import lax
from jax.experimental import pallas as pl
from jax.experimental.pallas import tpu as pltpu
from jax.experimental.pallas import tpu_sc as plsc

_SMOOTH = 1e-05
_NC, _NS, _L = 2, 16, 16
_NW = _NC * _NS

_K1 = 0.6931471805599453 / (1 << 23)
_G = (-88.02955, 3.6034518e-08, -6.5948397e-15, 3.6661022e-22, -1.1079349e-29)


def _ln(v):
    bits = plsc.bitcast(v, jnp.int32)
    y1 = bits.astype(jnp.float32) * jnp.float32(_K1)
    mant = lax.bitwise_and(bits, 0x007FFFFF).astype(jnp.float32)
    p = jnp.float32(_G[4])
    for k in (3, 2, 1, 0):
        p = p * mant + jnp.float32(_G[k])
    return y1 + p


def _sc_body(rows_w, rch, logit_hbm, tgt_hbm, abits_hbm, loss_hbm, cnt_hbm,
             ch_v, t_v, abits_vm, out_v, sem_a, sem_b):
    wid = lax.axis_index("s") * _NC + lax.axis_index("c")
    H, W = tgt_hbm.shape[1], tgt_hbm.shape[2]
    C = logit_hbm.shape[1]
    nch = rows_w // rch
    sub_per_batch = H // rows_w
    b = wid // sub_per_batch
    row0 = (wid % sub_per_batch) * rows_w
    sems = (sem_a, sem_b)

    pltpu.sync_copy(abits_hbm, abits_vm)
    abits_v = abits_vm[...]

    def copies(chunk, slot):
        r = row0 + chunk * rch
        srcs = [logit_hbm.at[b, c, pl.ds(r, rch), :] for c in range(C)]
        srcs.append(tgt_hbm.at[b, pl.ds(r, rch), :])
        dsts = [ch_v.at[slot, c] for c in range(C)]
        dsts.append(t_v.at[slot])
        return [(s, d, sem_a) for s, d in zip(srcs, dsts)]

    def issue(chunk, slot):
        for s, d, sem in copies(chunk, slot):
            pltpu.async_copy(s, d, sem)

    def drain(chunk, slot):
        for s, d, sem in copies(chunk, slot):
            pltpu.make_async_copy(s, d, sem).wait()

    unroll = 4

    def compute(slot, acc, cnt):
        lane = lax.iota(jnp.int32, _L)
        tsl = t_v.at[slot]
        csl = ch_v.at[slot]

        def row_body(i, rcarry):
            acc, cnt = rcarry
            rowsplat = jnp.full((_L,), i, jnp.int32)

            def col_body(j, ccarry):
                acc, cnt = ccarry
                for u in range(unroll):
                    col0 = (j * unroll + u) * _L
                    t = tsl[i, pl.ds(col0, _L)]
                    v = plsc.load_gather(csl, [t, rowsplat, lane + col0])
                    v = jnp.maximum(v, jnp.float32(_SMOOTH))
                    ln_v = _ln(v)
                    a = lax.bitwise_and(
                        lax.shift_right_logical(abits_v, t), 1
                    ).astype(jnp.float32)
                    acc = acc + a * (ln_v + jnp.float32(_SMOOTH))
                    cnt = cnt + jnp.minimum(t, 1)
                return acc, cnt

            return lax.fori_loop(0, W // (_L * unroll), col_body, (acc, cnt))

        return lax.fori_loop(0, rch, row_body, (acc, cnt))

    acc = jnp.zeros((_L,), jnp.float32)
    cnt = jnp.zeros((_L,), jnp.int32)
    issue(0, 0)

    def chunk_body(k, carry):
        acc, cnt = carry
        slot = lax.rem(k, 2)
        drain(k, slot)

        @pl.when(k + 1 < nch)
        def _():
            issue(k + 1, 1 - slot)

        return compute(slot, acc, cnt)

    acc, cnt = lax.fori_loop(0, nch, chunk_body, (acc, cnt))

    out_v[...] = acc
    pltpu.sync_copy(out_v, loss_hbm.at[pl.ds(wid * _L, _L)])
    out_v[...] = cnt.astype(jnp.float32)
    pltpu.sync_copy(out_v, cnt_hbm.at[pl.ds(wid * _L, _L)])


def kernel(logit, target, class_for_batch):
    B, C, H, W = logit.shape
    n = B * H * W
    rows_w = (B * H) // _NW
    rch = 16

    present = (jnp.arange(C)[:, None] == class_for_batch[None, :]).any(axis=1)
    alpha = jnp.where(present, 1.0, 0.0).astype(jnp.float32)
    alpha = alpha.at[0].set(0.0)
    abits = jnp.sum(
        jnp.where(alpha > 0, (1 << jnp.arange(C)).astype(jnp.int32), 0)
    ).astype(jnp.int32)
    abits16 = jnp.full((_L,), abits, jnp.int32)

    tg = target.reshape(B, H, W)

    mesh = plsc.VectorSubcoreMesh(
        core_axis_name="c", subcore_axis_name="s", num_cores=_NC, num_subcores=_NS
    )
    loss_part, cnt_part = pl.kernel(
        functools.partial(_sc_body, rows_w, rch),
        out_type=[
            jax.ShapeDtypeStruct((_NW * _L,), jnp.float32),
            jax.ShapeDtypeStruct((_NW * _L,), jnp.float32),
        ],
        mesh=mesh,
        compiler_params=pltpu.CompilerParams(
            use_tc_tiling_on_sc=True, needs_layout_passes=False
        ),
        scratch_types=[
            pltpu.VMEM((2, C, rch, W), jnp.float32),
            pltpu.VMEM((2, rch, W), jnp.int32),
            pltpu.VMEM((_L,), jnp.int32),
            pltpu.VMEM((_L,), jnp.float32),
            pltpu.SemaphoreType.DMA,
            pltpu.SemaphoreType.DMA,
        ],
    )(logit, tg, abits16)

    s = -jnp.sum(loss_part)
    pos = jnp.sum(cnt_part)
    return jnp.where(pos > 0, s / pos, s / jnp.float32(n))
